# baseline (speedup 1.0000x reference)
import functools

import jax
import jax.numpy as jnp
import numpy as np
from jax import lax
from jax.experimental import pallas as pl
from jax.experimental.pallas import tpu as pltpu
from jax.experimental.pallas import tpu_sc as plsc

N_POINTS = 10000
N_VIEWS = 3
SCALES = (56, 28, 14, 7)
CHANNELS = (64, 128, 256, 512)
C_TOTAL = 960
OUT_COLS = 3 + 3 * C_TOTAL
CHUNK = 16
N_CHUNKS = N_POINTS // CHUNK
G_ROWS = N_VIEWS * CHUNK

NUM_CORES = 2
NUM_SUBCORES = 16
NUM_WORKERS = NUM_CORES * NUM_SUBCORES
BASE_CHUNKS = N_CHUNKS // NUM_WORKERS
EXTRA = N_CHUNKS - BASE_CHUNKS * NUM_WORKERS
MAX_CHUNKS = BASE_CHUNKS + 1
PAD_CHUNKS = NUM_WORKERS * MAX_CHUNKS

CH_OFF = (0, 64, 192, 448)


def _normal(v):
    return v / jnp.linalg.norm(v)


def _camera_mat(param):
    theta = param[0] * np.pi / 180.0
    camy = param[3] * jnp.sin(param[1] * np.pi / 180.0)
    lens = param[3] * jnp.cos(param[1] * np.pi / 180.0)
    camx = lens * jnp.cos(theta)
    camz = lens * jnp.sin(theta)
    Z = jnp.stack([camx, camy, camz])
    x = camy * jnp.cos(theta + np.pi)
    z = camy * jnp.sin(theta + np.pi)
    Y = jnp.stack([x, lens, z])
    X = jnp.cross(Y, Z)
    cm_mat = jnp.stack([_normal(X), _normal(Y), _normal(Z)])
    return cm_mat, Z


def _camera_trans(param, xyz):
    c, o = _camera_mat(param)
    return (xyz - o) @ c.T


def _camera_trans_inv(param, xyz):
    c, o = _camera_mat(param)
    return xyz @ jnp.linalg.inv(c.T) + o


def _flat_indices(inputs, cameras):
    flat = [[] for _ in SCALES]
    for i in range(N_VIEWS):
        point_origin = _camera_trans_inv(cameras[0], inputs)
        point_current = _camera_trans(cameras[i], point_origin)
        X = point_current[:, 0]
        Y = point_current[:, 1]
        Z = point_current[:, 2]
        h = 248.0 * ((-Y) / (-Z)) + 112.0
        w = 248.0 * (X / (-Z)) + 112.0
        h = jnp.minimum(jnp.maximum(h, 0.0), 223.0)
        w = jnp.minimum(jnp.maximum(w, 0.0), 223.0)
        n = jnp.full(h.shape, float(i), dtype=jnp.float32)
        indeces = jnp.stack([n, h, w], 1)
        for j, s in enumerate(SCALES):
            idx = (indeces / (224.0 / float(s))).astype(jnp.int32)
            flat[j].append((idx[:, 0] * s + idx[:, 1]) * s + idx[:, 2])
    packed = []
    for j in range(len(SCALES)):
        a = jnp.stack(flat[j], 0)
        a = a.reshape(N_VIEWS, N_CHUNKS, CHUNK).transpose(1, 0, 2)
        packed.append(a.reshape(N_CHUNKS, G_ROWS))
    out = jnp.stack(packed, 1)
    return jnp.pad(out, ((0, PAD_CHUNKS - N_CHUNKS), (0, 0), (0, 0)))


def _sc_body(t0, t1, t2, t3, idxp, coords, out,
             idxall, coordall, r0, r1, r2, r3, outbuf,
             sg0, sg1, sg2, sg3, so):
    tables = (t0, t1, t2, t3)
    rows = (r0, r1, r2, r3)
    sg = (sg0, sg1, sg2, sg3)
    wid = lax.axis_index("s") * NUM_CORES + lax.axis_index("c")
    start = wid * BASE_CHUNKS + lax.min(wid, EXTRA)
    count = BASE_CHUNKS + jnp.where(wid < EXTRA, 1, 0)

    pltpu.sync_copy(idxp.at[pl.ds(start, MAX_CHUNKS)], idxall)
    pltpu.sync_copy(coords.at[pl.ds(start, MAX_CHUNKS)], coordall)

    third = jnp.float32(1.0 / 3.0)
    lanes = lax.broadcasted_iota(jnp.int32, (CHUNK,), 0)

    def issue_gathers(t):
        for j in range(4):
            pltpu.async_copy(
                tables[j].at[idxall.at[t, j]], rows[j], sg[j])

    def wait_gathers(t):
        for j in range(4):
            pltpu.make_async_copy(
                tables[j].at[idxall.at[t, j]], rows[j], sg[j]).wait()

    def out_desc(t):
        return pltpu.make_async_copy(
            outbuf, out.at[:, pl.ds((start + t) * CHUNK, CHUNK)], so)

    def chunk_body(t, carry):
        plsc.subcore_barrier()

        @pl.when(t < count)
        def _work():
            do_chunk(t)
        return carry

    def do_chunk(t):
        issue_gathers(t)

        @pl.when(t >= 1)
        def _drain_out():
            out_desc(t - 1).wait()

        wait_gathers(t)

        for d in range(3):
            outbuf[d, pl.ds(0, CHUNK)] = coordall[t, d, :]

        for j in range(4):
            cj = CHANNELS[j]

            @plsc.parallel_loop(0, cj, step=4)
            def _ch_loop(cb, _rows=rows[j], _coff=CH_OFF[j]):
                xs = []
                for dc in range(4):
                    cvec = jnp.full((CHUNK,), dc, jnp.int32) + cb
                    xs.append((
                        plsc.load_gather(_rows, [lanes, cvec]),
                        plsc.load_gather(_rows, [lanes + CHUNK, cvec]),
                        plsc.load_gather(_rows, [lanes + 2 * CHUNK, cvec])))
                for dc, (x0, x1, x2) in enumerate(xs):
                    mx = lax.max(lax.max(x0, x1), x2)
                    mn = (x0 + x1 + x2) * third
                    d0 = x0 - mn
                    d1 = x1 - mn
                    d2 = x2 - mn
                    var = (d0 * d0 + d1 * d1 + d2 * d2) * third
                    varc = lax.max(var, jnp.float32(1e-35))
                    iy = jnp.int32(0x5F3759DF) - lax.shift_right_logical(
                        lax.bitcast_convert_type(varc, jnp.int32), 1)
                    y = lax.bitcast_convert_type(iy, jnp.float32)
                    hv = varc * jnp.float32(0.5)
                    y = y * (jnp.float32(1.5) - hv * y * y)
                    y = y * (jnp.float32(1.5) - hv * y * y)
                    sd = varc * y
                    row = 3 + _coff + cb + dc
                    outbuf[row, pl.ds(0, CHUNK)] = mx
                    outbuf[row + C_TOTAL, pl.ds(0, CHUNK)] = mn
                    outbuf[row + 2 * C_TOTAL, pl.ds(0, CHUNK)] = sd

        pltpu.async_copy(
            outbuf, out.at[:, pl.ds((start + t) * CHUNK, CHUNK)], so)

    lax.fori_loop(0, MAX_CHUNKS, chunk_body, 0)
    out_desc(count - 1).wait()


def kernel(inputs, img_feat_0, img_feat_1, img_feat_2, img_feat_3, cameras):
    idxp = _flat_indices(inputs, cameras)
    coords = jnp.pad(inputs.reshape(N_CHUNKS, CHUNK, 3),
                     ((0, PAD_CHUNKS - N_CHUNKS), (0, 0), (0, 0)))
    coords = coords.transpose(0, 2, 1)
    feats = (img_feat_0, img_feat_1, img_feat_2, img_feat_3)
    tables = [f.reshape(N_VIEWS * s * s, c)
              for f, s, c in zip(feats, SCALES, CHANNELS)]

    mesh = plsc.VectorSubcoreMesh(core_axis_name="c", subcore_axis_name="s")
    run = functools.partial(
        pl.kernel,
        out_type=jax.ShapeDtypeStruct((OUT_COLS, N_POINTS), jnp.float32),
        mesh=mesh,
        compiler_params=pltpu.CompilerParams(use_tc_tiling_on_sc=False,
                                             needs_layout_passes=False),
        scratch_types=[
            pltpu.VMEM((MAX_CHUNKS, 4, G_ROWS), jnp.int32),
            pltpu.VMEM((MAX_CHUNKS, 3, CHUNK), jnp.float32),
            pltpu.VMEM((G_ROWS, CHANNELS[0]), jnp.float32),
            pltpu.VMEM((G_ROWS, CHANNELS[1]), jnp.float32),
            pltpu.VMEM((G_ROWS, CHANNELS[2]), jnp.float32),
            pltpu.VMEM((G_ROWS, CHANNELS[3]), jnp.float32),
            pltpu.VMEM((OUT_COLS, CHUNK), jnp.float32),
        ] + [pltpu.SemaphoreType.DMA] * 5,
    )(_sc_body)
    out = run(tables[0], tables[1], tables[2], tables[3], idxp, coords)
    return out.T


# speedup vs baseline: 1.1687x; 1.1687x over previous
import functools

import jax
import jax.numpy as jnp
import numpy as np
from jax import lax
from jax.experimental import pallas as pl
from jax.experimental.pallas import tpu as pltpu
from jax.experimental.pallas import tpu_sc as plsc

N_POINTS = 10000
N_VIEWS = 3
SCALES = (56, 28, 14, 7)
CHANNELS = (64, 128, 256, 512)
C_TOTAL = 960
OUT_COLS = 3 + 3 * C_TOTAL
CHUNK = 16
N_CHUNKS = N_POINTS // CHUNK
G_ROWS = N_VIEWS * CHUNK

NUM_CORES = 2
NUM_SUBCORES = 16
NUM_WORKERS = NUM_CORES * NUM_SUBCORES
BASE_CHUNKS = N_CHUNKS // NUM_WORKERS
EXTRA = N_CHUNKS - BASE_CHUNKS * NUM_WORKERS
MAX_CHUNKS = BASE_CHUNKS + 1
PAD_CHUNKS = NUM_WORKERS * MAX_CHUNKS

CH_OFF = (0, 64, 192, 448)


def _normal(v):
    return v / jnp.linalg.norm(v)


def _camera_mat(param):
    theta = param[0] * np.pi / 180.0
    camy = param[3] * jnp.sin(param[1] * np.pi / 180.0)
    lens = param[3] * jnp.cos(param[1] * np.pi / 180.0)
    camx = lens * jnp.cos(theta)
    camz = lens * jnp.sin(theta)
    Z = jnp.stack([camx, camy, camz])
    x = camy * jnp.cos(theta + np.pi)
    z = camy * jnp.sin(theta + np.pi)
    Y = jnp.stack([x, lens, z])
    X = jnp.cross(Y, Z)
    cm_mat = jnp.stack([_normal(X), _normal(Y), _normal(Z)])
    return cm_mat, Z


def _camera_trans(param, xyz):
    c, o = _camera_mat(param)
    return (xyz - o) @ c.T


def _camera_trans_inv(param, xyz):
    c, o = _camera_mat(param)
    return xyz @ jnp.linalg.inv(c.T) + o


def _flat_indices(inputs, cameras):
    flat = [[] for _ in SCALES]
    for i in range(N_VIEWS):
        point_origin = _camera_trans_inv(cameras[0], inputs)
        point_current = _camera_trans(cameras[i], point_origin)
        X = point_current[:, 0]
        Y = point_current[:, 1]
        Z = point_current[:, 2]
        h = 248.0 * ((-Y) / (-Z)) + 112.0
        w = 248.0 * (X / (-Z)) + 112.0
        h = jnp.minimum(jnp.maximum(h, 0.0), 223.0)
        w = jnp.minimum(jnp.maximum(w, 0.0), 223.0)
        n = jnp.full(h.shape, float(i), dtype=jnp.float32)
        indeces = jnp.stack([n, h, w], 1)
        for j, s in enumerate(SCALES):
            idx = (indeces / (224.0 / float(s))).astype(jnp.int32)
            flat[j].append((idx[:, 0] * s + idx[:, 1]) * s + idx[:, 2])
    packed = []
    for j in range(len(SCALES)):
        a = jnp.stack(flat[j], 0)
        a = a.reshape(N_VIEWS, N_CHUNKS, CHUNK).transpose(1, 0, 2)
        packed.append(a.reshape(N_CHUNKS, G_ROWS))
    out = jnp.stack(packed, 1)
    return jnp.pad(out, ((0, PAD_CHUNKS - N_CHUNKS), (0, 0), (0, 0)))


def _sc_body(t0, t1, t2, t3, idxp, coords, out,
             idxall, coordall, r0, r1, r2, r3, outbuf,
             sg0, sg1, sg2, sg3, so):
    tables = (t0, t1, t2, t3)
    rows = (r0, r1, r2, r3)
    sg = (sg0, sg1, sg2, sg3)
    wid = lax.axis_index("s") * NUM_CORES + lax.axis_index("c")
    start = wid * BASE_CHUNKS + lax.min(wid, EXTRA)
    count = BASE_CHUNKS + jnp.where(wid < EXTRA, 1, 0)

    pltpu.sync_copy(idxp.at[pl.ds(start, MAX_CHUNKS)], idxall)
    pltpu.sync_copy(coords.at[pl.ds(start, MAX_CHUNKS)], coordall)

    third = jnp.float32(1.0 / 3.0)
    lanes = lax.broadcasted_iota(jnp.int32, (CHUNK,), 0)

    def issue_gathers(t):
        for j in range(4):
            pltpu.async_copy(
                tables[j].at[idxall.at[t, j]], rows[j], sg[j])

    def wait_gathers(t):
        for j in range(4):
            pltpu.make_async_copy(
                tables[j].at[idxall.at[t, j]], rows[j], sg[j]).wait()

    def out_desc(t):
        return pltpu.make_async_copy(
            outbuf, out.at[:, pl.ds((start + t) * CHUNK, CHUNK)], so)

    def chunk_body(t, carry):
        do_chunk(t)
        return carry

    def do_chunk(t):
        issue_gathers(t)

        @pl.when(t >= 1)
        def _drain_out():
            out_desc(t - 1).wait()

        wait_gathers(t)

        for d in range(3):
            outbuf[d, pl.ds(0, CHUNK)] = coordall[t, d, :]

        for j in range(4):
            cj = CHANNELS[j]

            @plsc.parallel_loop(0, cj, step=8)
            def _ch_loop(cb, _rows=rows[j], _coff=CH_OFF[j]):
                cvec0 = jnp.full((CHUNK,), 0, jnp.int32) + cb
                xs = []
                for dc in range(8):
                    cvec = cvec0 + dc
                    xs.append((
                        plsc.load_gather(_rows, [lanes, cvec]),
                        plsc.load_gather(_rows, [lanes + CHUNK, cvec]),
                        plsc.load_gather(_rows, [lanes + 2 * CHUNK, cvec])))
                for dc, (x0, x1, x2) in enumerate(xs):
                    mx = lax.max(lax.max(x0, x1), x2)
                    mn = (x0 + x1 + x2) * third
                    d0 = x0 - mn
                    d1 = x1 - mn
                    d2 = x2 - mn
                    var = (d0 * d0 + d1 * d1 + d2 * d2) * third
                    varc = lax.max(var, jnp.float32(1e-35))
                    iy = jnp.int32(0x5F3759DF) - lax.shift_right_logical(
                        lax.bitcast_convert_type(varc, jnp.int32), 1)
                    y = lax.bitcast_convert_type(iy, jnp.float32)
                    hv = varc * jnp.float32(0.5)
                    y = y * (jnp.float32(1.5) - hv * y * y)
                    y = y * (jnp.float32(1.5) - hv * y * y)
                    sd = varc * y
                    row = 3 + _coff + cb + dc
                    outbuf[row, pl.ds(0, CHUNK)] = mx
                    outbuf[row + C_TOTAL, pl.ds(0, CHUNK)] = mn
                    outbuf[row + 2 * C_TOTAL, pl.ds(0, CHUNK)] = sd

        pltpu.async_copy(
            outbuf, out.at[:, pl.ds((start + t) * CHUNK, CHUNK)], so)

    lax.fori_loop(0, count, chunk_body, 0)
    out_desc(count - 1).wait()


def kernel(inputs, img_feat_0, img_feat_1, img_feat_2, img_feat_3, cameras):
    idxp = _flat_indices(inputs, cameras)
    coords = jnp.pad(inputs.reshape(N_CHUNKS, CHUNK, 3),
                     ((0, PAD_CHUNKS - N_CHUNKS), (0, 0), (0, 0)))
    coords = coords.transpose(0, 2, 1)
    feats = (img_feat_0, img_feat_1, img_feat_2, img_feat_3)
    tables = [f.reshape(N_VIEWS * s * s, c)
              for f, s, c in zip(feats, SCALES, CHANNELS)]

    mesh = plsc.VectorSubcoreMesh(core_axis_name="c", subcore_axis_name="s")
    run = functools.partial(
        pl.kernel,
        out_type=jax.ShapeDtypeStruct((OUT_COLS, N_POINTS), jnp.float32),
        mesh=mesh,
        compiler_params=pltpu.CompilerParams(use_tc_tiling_on_sc=False,
                                             needs_layout_passes=False),
        scratch_types=[
            pltpu.VMEM((MAX_CHUNKS, 4, G_ROWS), jnp.int32),
            pltpu.VMEM((MAX_CHUNKS, 3, CHUNK), jnp.float32),
            pltpu.VMEM((G_ROWS, CHANNELS[0]), jnp.float32),
            pltpu.VMEM((G_ROWS, CHANNELS[1]), jnp.float32),
            pltpu.VMEM((G_ROWS, CHANNELS[2]), jnp.float32),
            pltpu.VMEM((G_ROWS, CHANNELS[3]), jnp.float32),
            pltpu.VMEM((OUT_COLS, CHUNK), jnp.float32),
        ] + [pltpu.SemaphoreType.DMA] * 5,
    )(_sc_body)
    out = run(tables[0], tables[1], tables[2], tables[3], idxp, coords)
    return out.T


# speedup vs baseline: 1.2982x; 1.1108x over previous
import functools

import jax
import jax.numpy as jnp
import numpy as np
from jax import lax
from jax.experimental import pallas as pl
from jax.experimental.pallas import tpu as pltpu
from jax.experimental.pallas import tpu_sc as plsc

N_POINTS = 10000
N_VIEWS = 3
SCALES = (56, 28, 14, 7)
CHANNELS = (64, 128, 256, 512)
C_TOTAL = 960
OUT_COLS = 3 + 3 * C_TOTAL
CHUNK = 16
N_CHUNKS = N_POINTS // CHUNK
G_ROWS = N_VIEWS * CHUNK

NUM_CORES = 2
NUM_SUBCORES = 16
NUM_WORKERS = NUM_CORES * NUM_SUBCORES
BASE_CHUNKS = N_CHUNKS // NUM_WORKERS
EXTRA = N_CHUNKS - BASE_CHUNKS * NUM_WORKERS
MAX_CHUNKS = BASE_CHUNKS + 1
PAD_CHUNKS = NUM_WORKERS * MAX_CHUNKS

CH_OFF = (0, 64, 192, 448)


def _normal(v):
    return v / jnp.linalg.norm(v)


def _camera_mat(param):
    theta = param[0] * np.pi / 180.0
    camy = param[3] * jnp.sin(param[1] * np.pi / 180.0)
    lens = param[3] * jnp.cos(param[1] * np.pi / 180.0)
    camx = lens * jnp.cos(theta)
    camz = lens * jnp.sin(theta)
    Z = jnp.stack([camx, camy, camz])
    x = camy * jnp.cos(theta + np.pi)
    z = camy * jnp.sin(theta + np.pi)
    Y = jnp.stack([x, lens, z])
    X = jnp.cross(Y, Z)
    cm_mat = jnp.stack([_normal(X), _normal(Y), _normal(Z)])
    return cm_mat, Z


def _camera_trans(param, xyz):
    c, o = _camera_mat(param)
    return (xyz - o) @ c.T


def _camera_trans_inv(param, xyz):
    c, o = _camera_mat(param)
    return xyz @ jnp.linalg.inv(c.T) + o


def _flat_indices(inputs, cameras):
    flat = [[] for _ in SCALES]
    for i in range(N_VIEWS):
        point_origin = _camera_trans_inv(cameras[0], inputs)
        point_current = _camera_trans(cameras[i], point_origin)
        X = point_current[:, 0]
        Y = point_current[:, 1]
        Z = point_current[:, 2]
        h = 248.0 * ((-Y) / (-Z)) + 112.0
        w = 248.0 * (X / (-Z)) + 112.0
        h = jnp.minimum(jnp.maximum(h, 0.0), 223.0)
        w = jnp.minimum(jnp.maximum(w, 0.0), 223.0)
        n = jnp.full(h.shape, float(i), dtype=jnp.float32)
        indeces = jnp.stack([n, h, w], 1)
        for j, s in enumerate(SCALES):
            idx = (indeces / (224.0 / float(s))).astype(jnp.int32)
            flat[j].append((idx[:, 0] * s + idx[:, 1]) * s + idx[:, 2])
    packed = []
    for j in range(len(SCALES)):
        a = jnp.stack(flat[j], 0)
        a = a.reshape(N_VIEWS, N_CHUNKS, CHUNK).transpose(1, 0, 2)
        packed.append(a.reshape(N_CHUNKS, G_ROWS))
    out = jnp.stack(packed, 1)
    return jnp.pad(out, ((0, PAD_CHUNKS - N_CHUNKS), (0, 0), (0, 0)))


def _sc_body(t0, t1, t2, t3, idxp, coords, out,
             idxall, coordall, r0, r1, r2, r3, ob0, ob1, ob2, ob3,
             sg0, sg1, sg2, sg3, so0, so1, so2, so3):
    tables = (t0, t1, t2, t3)
    rows = (r0, r1, r2, r3)
    obufs = (ob0, ob1, ob2, ob3)
    sg = (sg0, sg1, sg2, sg3)
    so = (so0, so1, so2, so3)
    wid = lax.axis_index("s") * NUM_CORES + lax.axis_index("c")
    start = wid * BASE_CHUNKS + lax.min(wid, EXTRA)
    count = BASE_CHUNKS + jnp.where(wid < EXTRA, 1, 0)

    pltpu.sync_copy(idxp.at[pl.ds(start, MAX_CHUNKS)], idxall)
    pltpu.sync_copy(coords.at[pl.ds(start, MAX_CHUNKS)], coordall)

    third = jnp.float32(1.0 / 3.0)
    lanes = lax.broadcasted_iota(jnp.int32, (CHUNK,), 0)

    def gather_desc(t, j):
        return pltpu.make_async_copy(
            tables[j].at[idxall.at[t, j]], rows[j], sg[j])

    def out_descs(t, j):
        cj = CHANNELS[j]
        lead = 3 if j == 0 else 0
        pt = (start + t) * CHUNK
        return [
            pltpu.make_async_copy(
                obufs[j].at[pl.ds(0, lead + cj)],
                out.at[pl.ds(3 + CH_OFF[j] - lead, lead + cj),
                       pl.ds(pt, CHUNK)], so[j]),
            pltpu.make_async_copy(
                obufs[j].at[pl.ds(lead + cj, cj)],
                out.at[pl.ds(3 + C_TOTAL + CH_OFF[j], cj),
                       pl.ds(pt, CHUNK)], so[j]),
            pltpu.make_async_copy(
                obufs[j].at[pl.ds(lead + 2 * cj, cj)],
                out.at[pl.ds(3 + 2 * C_TOTAL + CH_OFF[j], cj),
                       pl.ds(pt, CHUNK)], so[j]),
        ]

    def compute_scale(t, j):
        cj = CHANNELS[j]
        lead = 3 if j == 0 else 0
        gather_desc(t, j).wait()

        @pl.when(t >= 1)
        def _drain_out():
            for d in out_descs(t - 1, j):
                d.wait()

        if j == 0:
            for d in range(3):
                obufs[0][d, pl.ds(0, CHUNK)] = coordall[t, d, :]

        @plsc.parallel_loop(0, cj, step=8)
        def _ch_loop(cb, _rows=rows[j], _obuf=obufs[j], _lead=lead, _cj=cj):
            cvec0 = jnp.full((CHUNK,), 0, jnp.int32) + cb
            xs = []
            for dc in range(8):
                cvec = cvec0 + dc
                xs.append((
                    plsc.load_gather(_rows, [lanes, cvec]),
                    plsc.load_gather(_rows, [lanes + CHUNK, cvec]),
                    plsc.load_gather(_rows, [lanes + 2 * CHUNK, cvec])))
            for dc, (x0, x1, x2) in enumerate(xs):
                mx = lax.max(lax.max(x0, x1), x2)
                mn = (x0 + x1 + x2) * third
                d0 = x0 - mn
                d1 = x1 - mn
                d2 = x2 - mn
                var = (d0 * d0 + d1 * d1 + d2 * d2) * third
                varc = lax.max(var, jnp.float32(1e-35))
                iy = jnp.int32(0x5F3759DF) - lax.shift_right_logical(
                    lax.bitcast_convert_type(varc, jnp.int32), 1)
                y = lax.bitcast_convert_type(iy, jnp.float32)
                hv = varc * jnp.float32(0.5)
                y = y * (jnp.float32(1.5) - hv * y * y)
                y = y * (jnp.float32(1.5) - hv * y * y)
                sd = varc * y
                row = _lead + cb + dc
                _obuf[row, pl.ds(0, CHUNK)] = mx
                _obuf[row + _cj, pl.ds(0, CHUNK)] = mn
                _obuf[row + 2 * _cj, pl.ds(0, CHUNK)] = sd

        for d in out_descs(t, j):
            d.start()

        @pl.when(t + 1 < count)
        def _prefetch():
            gather_desc(t + 1, j).start()

    def chunk_body(t, carry):
        for j in range(4):
            compute_scale(t, j)
        return carry

    for j in range(4):
        gather_desc(0, j).start()
    lax.fori_loop(0, count, chunk_body, 0)
    for j in range(4):
        for d in out_descs(count - 1, j):
            d.wait()


def kernel(inputs, img_feat_0, img_feat_1, img_feat_2, img_feat_3, cameras):
    idxp = _flat_indices(inputs, cameras)
    coords = jnp.pad(inputs.reshape(N_CHUNKS, CHUNK, 3),
                     ((0, PAD_CHUNKS - N_CHUNKS), (0, 0), (0, 0)))
    coords = coords.transpose(0, 2, 1)
    feats = (img_feat_0, img_feat_1, img_feat_2, img_feat_3)
    tables = [f.reshape(N_VIEWS * s * s, c)
              for f, s, c in zip(feats, SCALES, CHANNELS)]

    mesh = plsc.VectorSubcoreMesh(core_axis_name="c", subcore_axis_name="s")
    run = functools.partial(
        pl.kernel,
        out_type=jax.ShapeDtypeStruct((OUT_COLS, N_POINTS), jnp.float32),
        mesh=mesh,
        compiler_params=pltpu.CompilerParams(use_tc_tiling_on_sc=False,
                                             needs_layout_passes=False),
        scratch_types=[
            pltpu.VMEM((MAX_CHUNKS, 4, G_ROWS), jnp.int32),
            pltpu.VMEM((MAX_CHUNKS, 3, CHUNK), jnp.float32),
            pltpu.VMEM((G_ROWS, CHANNELS[0]), jnp.float32),
            pltpu.VMEM((G_ROWS, CHANNELS[1]), jnp.float32),
            pltpu.VMEM((G_ROWS, CHANNELS[2]), jnp.float32),
            pltpu.VMEM((G_ROWS, CHANNELS[3]), jnp.float32),
            pltpu.VMEM((3 + 3 * CHANNELS[0], CHUNK), jnp.float32),
            pltpu.VMEM((3 * CHANNELS[1], CHUNK), jnp.float32),
            pltpu.VMEM((3 * CHANNELS[2], CHUNK), jnp.float32),
            pltpu.VMEM((3 * CHANNELS[3], CHUNK), jnp.float32),
        ] + [pltpu.SemaphoreType.DMA] * 8,
    )(_sc_body)
    out = run(tables[0], tables[1], tables[2], tables[3], idxp, coords)
    return out.T


# speedup vs baseline: 1.3147x; 1.0127x over previous
import functools

import jax
import jax.numpy as jnp
import numpy as np
from jax import lax
from jax.experimental import pallas as pl
from jax.experimental.pallas import tpu as pltpu
from jax.experimental.pallas import tpu_sc as plsc

N_POINTS = 10000
N_VIEWS = 3
SCALES = (56, 28, 14, 7)
CHANNELS = (64, 128, 256, 512)
C_TOTAL = 960
OUT_COLS = 3 + 3 * C_TOTAL
CHUNK = 16
N_CHUNKS = N_POINTS // CHUNK
G_ROWS = N_VIEWS * CHUNK

NUM_CORES = 2
NUM_SUBCORES = 16
NUM_WORKERS = NUM_CORES * NUM_SUBCORES
BASE_CHUNKS = N_CHUNKS // NUM_WORKERS
EXTRA = N_CHUNKS - BASE_CHUNKS * NUM_WORKERS
MAX_CHUNKS = BASE_CHUNKS + 1
PAD_CHUNKS = NUM_WORKERS * MAX_CHUNKS

CH_OFF = (0, 64, 192, 448)


def _normal(v):
    return v / jnp.linalg.norm(v)


def _camera_mat(param):
    theta = param[0] * np.pi / 180.0
    camy = param[3] * jnp.sin(param[1] * np.pi / 180.0)
    lens = param[3] * jnp.cos(param[1] * np.pi / 180.0)
    camx = lens * jnp.cos(theta)
    camz = lens * jnp.sin(theta)
    Z = jnp.stack([camx, camy, camz])
    x = camy * jnp.cos(theta + np.pi)
    z = camy * jnp.sin(theta + np.pi)
    Y = jnp.stack([x, lens, z])
    X = jnp.cross(Y, Z)
    cm_mat = jnp.stack([_normal(X), _normal(Y), _normal(Z)])
    return cm_mat, Z


def _camera_trans(param, xyz):
    c, o = _camera_mat(param)
    return (xyz - o) @ c.T


def _camera_trans_inv(param, xyz):
    c, o = _camera_mat(param)
    return xyz @ jnp.linalg.inv(c.T) + o


def _flat_indices(inputs, cameras):
    flat = [[] for _ in SCALES]
    for i in range(N_VIEWS):
        point_origin = _camera_trans_inv(cameras[0], inputs)
        point_current = _camera_trans(cameras[i], point_origin)
        X = point_current[:, 0]
        Y = point_current[:, 1]
        Z = point_current[:, 2]
        h = 248.0 * ((-Y) / (-Z)) + 112.0
        w = 248.0 * (X / (-Z)) + 112.0
        h = jnp.minimum(jnp.maximum(h, 0.0), 223.0)
        w = jnp.minimum(jnp.maximum(w, 0.0), 223.0)
        n = jnp.full(h.shape, float(i), dtype=jnp.float32)
        indeces = jnp.stack([n, h, w], 1)
        for j, s in enumerate(SCALES):
            idx = (indeces / (224.0 / float(s))).astype(jnp.int32)
            flat[j].append((idx[:, 0] * s + idx[:, 1]) * s + idx[:, 2])
    packed = []
    for j in range(len(SCALES)):
        a = jnp.stack(flat[j], 0)
        a = a.reshape(N_VIEWS, N_CHUNKS, CHUNK).transpose(1, 0, 2)
        packed.append(a.reshape(N_CHUNKS, G_ROWS))
    out = jnp.stack(packed, 1)
    return jnp.pad(out, ((0, PAD_CHUNKS - N_CHUNKS), (0, 0), (0, 0)))


def _sc_body(t0, t1, t2, t3, idxp, coords, out,
             idxall, coordall, r0, r1, r2, r3, ob0, ob1, ob2, ob3,
             sg0, sg1, sg2, sg3, so0, so1, so2, so3):
    tables = (t0, t1, t2, t3)
    rows = (r0, r1, r2, r3)
    obufs = (ob0, ob1, ob2, ob3)
    sg = (sg0, sg1, sg2, sg3)
    so = (so0, so1, so2, so3)
    wid = lax.axis_index("s") * NUM_CORES + lax.axis_index("c")
    start = wid * BASE_CHUNKS + lax.min(wid, EXTRA)
    count = BASE_CHUNKS + jnp.where(wid < EXTRA, 1, 0)

    pltpu.sync_copy(idxp.at[pl.ds(start, MAX_CHUNKS)], idxall)
    pltpu.sync_copy(coords.at[pl.ds(start, MAX_CHUNKS)], coordall)

    third = jnp.float32(1.0 / 3.0)
    lanes = lax.broadcasted_iota(jnp.int32, (CHUNK,), 0)

    def gather_desc(t, j):
        return pltpu.make_async_copy(
            tables[j].at[idxall.at[t, j]], rows[j], sg[j])

    def out_descs(t, j):
        cj = CHANNELS[j]
        lead = 3 if j == 0 else 0
        pt = (start + t) * CHUNK
        return [
            pltpu.make_async_copy(
                obufs[j].at[pl.ds(0, lead + cj)],
                out.at[pl.ds(3 + CH_OFF[j] - lead, lead + cj),
                       pl.ds(pt, CHUNK)], so[j]),
            pltpu.make_async_copy(
                obufs[j].at[pl.ds(lead + cj, cj)],
                out.at[pl.ds(3 + C_TOTAL + CH_OFF[j], cj),
                       pl.ds(pt, CHUNK)], so[j]),
            pltpu.make_async_copy(
                obufs[j].at[pl.ds(lead + 2 * cj, cj)],
                out.at[pl.ds(3 + 2 * C_TOTAL + CH_OFF[j], cj),
                       pl.ds(pt, CHUNK)], so[j]),
        ]

    def compute_scale(t, j):
        cj = CHANNELS[j]
        lead = 3 if j == 0 else 0
        gather_desc(t, j).wait()

        @pl.when(t >= 1)
        def _drain_out():
            for d in out_descs(t - 1, j):
                d.wait()

        if j == 0:
            for d in range(3):
                obufs[0][d, pl.ds(0, CHUNK)] = coordall[t, d, :]

        @plsc.parallel_loop(0, cj, step=8)
        def _ch_loop(cb, _rows=rows[j], _obuf=obufs[j], _lead=lead, _cj=cj):
            cvec0 = jnp.full((CHUNK,), 0, jnp.int32) + cb
            xs = []
            for dc in range(8):
                cvec = cvec0 + dc
                xs.append((
                    plsc.load_gather(_rows, [lanes, cvec]),
                    plsc.load_gather(_rows, [lanes + CHUNK, cvec]),
                    plsc.load_gather(_rows, [lanes + 2 * CHUNK, cvec])))
            for dc, (x0, x1, x2) in enumerate(xs):
                mx = lax.max(lax.max(x0, x1), x2)
                mn = (x0 + x1 + x2) * third
                sq = (x0 * x0 + x1 * x1 + x2 * x2) * third
                var = sq - mn * mn
                varc = lax.max(var, jnp.float32(1e-35))
                iy = jnp.int32(0x5F3759DF) - lax.shift_right_logical(
                    lax.bitcast_convert_type(varc, jnp.int32), 1)
                y = lax.bitcast_convert_type(iy, jnp.float32)
                hv = varc * jnp.float32(0.5)
                y = y * (jnp.float32(1.5) - hv * y * y)
                y = y * (jnp.float32(1.5) - hv * y * y)
                sd = varc * y
                row = _lead + cb + dc
                _obuf[row, pl.ds(0, CHUNK)] = mx
                _obuf[row + _cj, pl.ds(0, CHUNK)] = mn
                _obuf[row + 2 * _cj, pl.ds(0, CHUNK)] = sd

        for d in out_descs(t, j):
            d.start()

        @pl.when(t + 1 < count)
        def _prefetch():
            gather_desc(t + 1, j).start()

    def chunk_body(t, carry):
        for j in range(4):
            compute_scale(t, j)
        return carry

    for j in range(4):
        gather_desc(0, j).start()
    lax.fori_loop(0, count, chunk_body, 0)
    for j in range(4):
        for d in out_descs(count - 1, j):
            d.wait()


def kernel(inputs, img_feat_0, img_feat_1, img_feat_2, img_feat_3, cameras):
    idxp = _flat_indices(inputs, cameras)
    coords = jnp.pad(inputs.reshape(N_CHUNKS, CHUNK, 3),
                     ((0, PAD_CHUNKS - N_CHUNKS), (0, 0), (0, 0)))
    coords = coords.transpose(0, 2, 1)
    feats = (img_feat_0, img_feat_1, img_feat_2, img_feat_3)
    tables = [f.reshape(N_VIEWS * s * s, c)
              for f, s, c in zip(feats, SCALES, CHANNELS)]

    mesh = plsc.VectorSubcoreMesh(core_axis_name="c", subcore_axis_name="s")
    run = functools.partial(
        pl.kernel,
        out_type=jax.ShapeDtypeStruct((OUT_COLS, N_POINTS), jnp.float32),
        mesh=mesh,
        compiler_params=pltpu.CompilerParams(use_tc_tiling_on_sc=False,
                                             needs_layout_passes=False),
        scratch_types=[
            pltpu.VMEM((MAX_CHUNKS, 4, G_ROWS), jnp.int32),
            pltpu.VMEM((MAX_CHUNKS, 3, CHUNK), jnp.float32),
            pltpu.VMEM((G_ROWS, CHANNELS[0]), jnp.float32),
            pltpu.VMEM((G_ROWS, CHANNELS[1]), jnp.float32),
            pltpu.VMEM((G_ROWS, CHANNELS[2]), jnp.float32),
            pltpu.VMEM((G_ROWS, CHANNELS[3]), jnp.float32),
            pltpu.VMEM((3 + 3 * CHANNELS[0], CHUNK), jnp.float32),
            pltpu.VMEM((3 * CHANNELS[1], CHUNK), jnp.float32),
            pltpu.VMEM((3 * CHANNELS[2], CHUNK), jnp.float32),
            pltpu.VMEM((3 * CHANNELS[3], CHUNK), jnp.float32),
        ] + [pltpu.SemaphoreType.DMA] * 8,
    )(_sc_body)
    out = run(tables[0], tables[1], tables[2], tables[3], idxp, coords)
    return out.T
